# Initial kernel scaffold; baseline (speedup 1.0000x reference)
import jax
import jax.numpy as jnp
from jax.experimental import pallas as pl


def kernel(node_coordinates, edge_lengths, edge_vectors, node_from, node_to, node_graph_index, num_nodes, num_graphs, bn_gamma, bn_beta, bn_mean, bn_var, W1, b1, W2, b2, W3, b3, W4, b4, Wo, bo):
    raise NotImplementedError("write your pallas kernel here")


# speedup vs baseline: 2.5878x; 2.5878x over previous
import functools

import jax
import jax.numpy as jnp
from jax import lax
from jax.experimental import pallas as pl
from jax.experimental.pallas import tpu as pltpu
from jax.experimental.pallas import tpu_sc as plsc

STATE = 20
NG = 500
NCORES = 2
NSUB = 16
NW = NCORES * NSUB


def _sc_mesh():
    return plsc.VectorSubcoreMesh(core_axis_name="c", subcore_axis_name="s")


def _round_up(x, m):
    return (x + m - 1) // m * m


def _pick_chunk(total, cap):
    c = min(total, max(cap, 8)) // 8 * 8
    while c > 8 and total % c != 0:
        c -= 8
    return c


def _pick_tile(total, cap):
    return _pick_chunk(total, cap)


def _gather_rows(table, idx, cap_bytes=400_000):
    M = idx.shape[0]
    _, D = table.shape
    per_w = M // NW
    max_rows = cap_bytes // (4 * (D + 1))
    chunk = _pick_chunk(per_w, max_rows)
    nch = per_w // chunk

    @functools.partial(
        pl.kernel,
        out_type=jax.ShapeDtypeStruct((M, D), jnp.float32),
        mesh=_sc_mesh(),
        scratch_types=[
            pltpu.VMEM((chunk,), jnp.int32),
            pltpu.VMEM((chunk, D), jnp.float32),
            pltpu.SemaphoreType.DMA,
        ],
        compiler_params=pltpu.CompilerParams(use_tc_tiling_on_sc=False),
    )
    def k(table_hbm, idx_hbm, out_hbm, idx_v, rows_v, sem):
        wid = lax.axis_index("s") * NCORES + lax.axis_index("c")
        base = wid * per_w

        def body(i, carry):
            off = base + i * chunk
            pltpu.sync_copy(idx_hbm.at[pl.ds(off, chunk)], idx_v)
            pltpu.async_copy(table_hbm.at[idx_v], rows_v, sem).wait()
            pltpu.sync_copy(rows_v, out_hbm.at[pl.ds(off, chunk)])
            return carry

        lax.fori_loop(0, nch, body, 0)

    return k(table, idx)


def _scatter_add_rows(values, idx, prev, cap_bytes=400_000):
    M, D = values.shape
    P = prev.shape[0]
    per_t = M // NSUB
    max_rows = cap_bytes // (4 * (D + 1))
    chunk = _pick_chunk(per_t, max_rows)
    nch = per_t // chunk
    rows_init = P // NSUB
    half = P // NCORES
    rows_out = half // NSUB

    @functools.partial(
        pl.kernel,
        out_type=jax.ShapeDtypeStruct((P, D), jnp.float32),
        mesh=_sc_mesh(),
        scratch_types=[
            pltpu.VMEM((chunk,), jnp.int32),
            pltpu.VMEM((chunk, D), jnp.float32),
            pltpu.VMEM_SHARED((P, D), jnp.float32),
            pltpu.SemaphoreType.DMA,
        ],
        compiler_params=pltpu.CompilerParams(use_tc_tiling_on_sc=False),
    )
    def k(vals_hbm, idx_hbm, prev_hbm, out_hbm, idx_v, val_v, accum, sem):
        cid = lax.axis_index("c")
        sid = lax.axis_index("s")
        r0 = sid * rows_init
        pltpu.sync_copy(prev_hbm.at[pl.ds(r0, rows_init)],
                        accum.at[pl.ds(r0, rows_init)])
        plsc.subcore_barrier()

        def body(i, carry):
            off = sid * per_t + i * chunk
            pltpu.sync_copy(idx_hbm.at[pl.ds(off, chunk)], idx_v)
            pltpu.sync_copy(vals_hbm.at[pl.ds(off, chunk)], val_v)
            pltpu.sync_copy(val_v, accum.at[idx_v], add=True)
            return carry

        lax.fori_loop(0, nch, body, 0)
        plsc.subcore_barrier()
        o0 = cid * half + sid * rows_out
        pltpu.sync_copy(accum.at[pl.ds(o0, rows_out)],
                        out_hbm.at[pl.ds(o0, rows_out)])

    return k(values, idx, prev)


def _wspec(shp):
    return pl.BlockSpec(shp, lambda i: (0, 0))


def _tc_round1(cf, ct, evl, W1b, b1t, W2, b2, W3, b3, W4, b4, T):
    E = cf.shape[0]

    def body(cf_ref, ct_ref, ev_ref, w1b, bb1, w2, bb2, w3, bb3, w4, bb4,
             msg_ref, ef_ref):
        cfv, ctv, evv = cf_ref[...], ct_ref[...], ev_ref[...]
        d = cfv - ctv
        ln = evv[:, 3:4]
        cd = jnp.sum(jnp.abs(d), axis=1, keepdims=True)
        dc = jnp.sum(cfv * ctv, axis=1, keepdims=True)
        dd = jnp.sum(d * evv, axis=1, keepdims=True)
        ef_ref[...] = jnp.concatenate([ln, cd, dc, dd], axis=1)
        z = (bb1[...] + ln * w1b[0:1, :] + cd * w1b[1:2, :]
             + dc * w1b[2:3, :] + dd * w1b[3:4, :])
        h = jnp.tanh(z)
        h = jnp.tanh(h @ w2[...] + bb2[...])
        h = jnp.tanh(h @ w3[...] + bb3[...])
        msg_ref[...] = jnp.tanh(h @ w4[...] + bb4[...])

    return pl.pallas_call(
        body,
        grid=(E // T,),
        in_specs=[
            pl.BlockSpec((T, 4), lambda i: (i, 0)),
            pl.BlockSpec((T, 4), lambda i: (i, 0)),
            pl.BlockSpec((T, 4), lambda i: (i, 0)),
            _wspec((4, 40)), _wspec((1, 40)),
            _wspec((40, 150)), _wspec((1, 150)),
            _wspec((150, 40)), _wspec((1, 40)),
            _wspec((40, STATE)), _wspec((1, STATE)),
        ],
        out_specs=[
            pl.BlockSpec((T, STATE), lambda i: (i, 0)),
            pl.BlockSpec((T, 4), lambda i: (i, 0)),
        ],
        out_shape=[
            jax.ShapeDtypeStruct((E, STATE), jnp.float32),
            jax.ShapeDtypeStruct((E, 4), jnp.float32),
        ],
        compiler_params=pltpu.CompilerParams(
            dimension_semantics=("parallel",)),
    )(cf, ct, evl, W1b, b1t, W2, b2, W3, b3, W4, b4)


def _tc_mlp(gathered, ef, W1a, W1b, b1t, W2, b2, W3, b3, W4, b4, T):
    E = gathered.shape[0]

    def body(g_ref, ef_ref, w1a, w1b, bb1, w2, bb2, w3, bb3, w4, bb4,
             out_ref):
        efv = ef_ref[...]
        z = g_ref[...] @ w1a[...] + bb1[...]
        z = (z + efv[:, 0:1] * w1b[0:1, :] + efv[:, 1:2] * w1b[1:2, :]
             + efv[:, 2:3] * w1b[2:3, :] + efv[:, 3:4] * w1b[3:4, :])
        h = jnp.tanh(z)
        h = jnp.tanh(h @ w2[...] + bb2[...])
        h = jnp.tanh(h @ w3[...] + bb3[...])
        out_ref[...] = jnp.tanh(h @ w4[...] + bb4[...])

    return pl.pallas_call(
        body,
        grid=(E // T,),
        in_specs=[
            pl.BlockSpec((T, STATE), lambda i: (i, 0)),
            pl.BlockSpec((T, 4), lambda i: (i, 0)),
            _wspec((STATE, 40)), _wspec((4, 40)), _wspec((1, 40)),
            _wspec((40, 150)), _wspec((1, 150)),
            _wspec((150, 40)), _wspec((1, 40)),
            _wspec((40, STATE)), _wspec((1, STATE)),
        ],
        out_specs=pl.BlockSpec((T, STATE), lambda i: (i, 0)),
        out_shape=jax.ShapeDtypeStruct((E, STATE), jnp.float32),
        compiler_params=pltpu.CompilerParams(
            dimension_semantics=("parallel",)),
    )(gathered, ef, W1a, W1b, b1t, W2, b2, W3, b3, W4, b4)


def _tc_outnet(gstate, Wo, bo):
    G = gstate.shape[0]

    def body(g_ref, wo, bo_, out_ref):
        ev = g_ref[...] @ wo[...] + bo_[...]
        mu = ev[:, 0:1]
        sg = ev[:, 1:2]
        sp = jnp.maximum(sg, 0.0) + jnp.log1p(jnp.exp(-jnp.abs(sg)))
        out_ref[...] = jnp.concatenate([mu, sp], axis=1)

    return pl.pallas_call(
        body,
        grid=(1,),
        in_specs=[
            pl.BlockSpec((G, STATE), lambda i: (0, 0)),
            _wspec((STATE, 2)), _wspec((1, 2)),
        ],
        out_specs=pl.BlockSpec((G, 2), lambda i: (0, 0)),
        out_shape=jax.ShapeDtypeStruct((G, 2), jnp.float32),
    )(gstate, Wo, bo)


def kernel(node_coordinates, edge_lengths, edge_vectors, node_from, node_to,
           node_graph_index, num_nodes, num_graphs,
           bn_gamma, bn_beta, bn_mean, bn_var,
           W1, b1, W2, b2, W3, b3, W4, b4, Wo, bo):
    E = node_from.shape[0]
    N = node_coordinates.shape[0]
    NP = _round_up(N, 128)
    G = _round_up(NG, 64)
    T = _pick_tile(E, 4096)

    nf = node_from.astype(jnp.int32)
    nt = node_to.astype(jnp.int32)
    gidx = node_graph_index.astype(jnp.int32)

    s = bn_gamma * lax.rsqrt(bn_var + 1e-5)
    t = bn_beta - bn_mean * s
    W1s = W1 * s[:, None]
    b1t = (b1 + t @ W1).reshape(1, -1)
    W1a, W1b = W1s[:STATE], W1s[STATE:]
    b2r, b3r, b4r = b2.reshape(1, -1), b3.reshape(1, -1), b4.reshape(1, -1)
    bor = bo.reshape(1, -1)

    coords4 = jnp.pad(node_coordinates, ((0, 0), (0, 1)))
    evl = jnp.concatenate([edge_vectors, edge_lengths], axis=1)
    both = _gather_rows(coords4, jnp.concatenate([nf, nt]))
    cf, ct = both[:E], both[E:]

    msg, ef = _tc_round1(cf, ct, evl, W1b, b1t, W2, b2r, W3, b3r, W4, b4r, T)

    state = jnp.zeros((NP, STATE), jnp.float32)
    for r in range(3):
        state = _scatter_add_rows(msg, nt, state)
        if r < 2:
            gathered = _gather_rows(state, nf)
            msg = _tc_mlp(gathered, ef, W1a, W1b, b1t,
                          W2, b2r, W3, b3r, W4, b4r, T)

    gidx_pad = jnp.concatenate(
        [gidx, jnp.full((NP - N,), G - 1, jnp.int32)])
    gstate = _scatter_add_rows(state, gidx_pad,
                               jnp.zeros((G, STATE), jnp.float32))

    out = _tc_outnet(gstate, Wo, bor)
    return out[:NG]


# speedup vs baseline: 2.9671x; 1.1466x over previous
import functools

import jax
import jax.numpy as jnp
from jax import lax
from jax.experimental import pallas as pl
from jax.experimental.pallas import tpu as pltpu
from jax.experimental.pallas import tpu_sc as plsc

STATE = 20
SD = 24
EFD = 8
NG = 500
NCORES = 2
NSUB = 16
NW = NCORES * NSUB


def _sc_mesh():
    return plsc.VectorSubcoreMesh(core_axis_name="c", subcore_axis_name="s")


def _round_up(x, m):
    return (x + m - 1) // m * m


def _pick_chunk(total, cap):
    c = min(total, max(cap, 8)) // 8 * 8
    while c > 8 and total % c != 0:
        c -= 8
    return c


def _gather_rows(table, idx):
    M = idx.shape[0]
    _, D = table.shape
    per_w = M // NW
    chunk = _pick_chunk(per_w, 440_000 // (8 * (D + 1)))
    nch = per_w // chunk

    @functools.partial(
        pl.kernel,
        out_type=jax.ShapeDtypeStruct((M, D), jnp.float32),
        mesh=_sc_mesh(),
        scratch_types=[
            pltpu.VMEM((chunk,), jnp.int32),
            pltpu.VMEM((chunk,), jnp.int32),
            pltpu.VMEM((chunk, D), jnp.float32),
            pltpu.VMEM((chunk, D), jnp.float32),
            pltpu.SemaphoreType.DMA,
            pltpu.SemaphoreType.DMA,
            pltpu.SemaphoreType.DMA,
        ],
        compiler_params=pltpu.CompilerParams(use_tc_tiling_on_sc=False),
    )
    def k(table_hbm, idx_hbm, out_hbm, idx_a, idx_b, rows_a, rows_b,
          isem, gsem, osem):
        wid = lax.axis_index("s") * NCORES + lax.axis_index("c")
        base = wid * per_w
        idx_bufs = [idx_a, idx_b]
        row_bufs = [rows_a, rows_b]
        idx_d = [None] * nch
        out_d = [None] * nch
        idx_d[0] = pltpu.async_copy(
            idx_hbm.at[pl.ds(base, chunk)], idx_bufs[0], isem)
        for i in range(nch):
            p = i % 2
            idx_d[i].wait()
            if i + 1 < nch:
                idx_d[i + 1] = pltpu.async_copy(
                    idx_hbm.at[pl.ds(base + (i + 1) * chunk, chunk)],
                    idx_bufs[(i + 1) % 2], isem)
            if i >= 2:
                out_d[i - 2].wait()
            pltpu.async_copy(
                table_hbm.at[idx_bufs[p]], row_bufs[p], gsem).wait()
            out_d[i] = pltpu.async_copy(
                row_bufs[p], out_hbm.at[pl.ds(base + i * chunk, chunk)],
                osem)
        if nch >= 2:
            out_d[nch - 2].wait()
        out_d[nch - 1].wait()

    return k(table, idx)


def _scatter_add_rows(values, idx, prev):
    M, D = values.shape
    P = prev.shape[0]
    per_t = M // NSUB
    chunk = _pick_chunk(per_t, 440_000 // (8 * (D + 1)))
    nch = per_t // chunk
    rows_init = P // NSUB
    half = P // NCORES
    rows_out = half // NSUB

    @functools.partial(
        pl.kernel,
        out_type=jax.ShapeDtypeStruct((P, D), jnp.float32),
        mesh=_sc_mesh(),
        scratch_types=[
            pltpu.VMEM((chunk,), jnp.int32),
            pltpu.VMEM((chunk, D), jnp.float32),
            pltpu.VMEM_SHARED((P, D), jnp.float32),
            pltpu.SemaphoreType.DMA,
        ],
        compiler_params=pltpu.CompilerParams(use_tc_tiling_on_sc=False),
    )
    def k(vals_hbm, idx_hbm, prev_hbm, out_hbm, idx_v, val_v, accum, sem):
        cid = lax.axis_index("c")
        sid = lax.axis_index("s")
        r0 = sid * rows_init
        pltpu.sync_copy(prev_hbm.at[pl.ds(r0, rows_init)],
                        accum.at[pl.ds(r0, rows_init)])
        plsc.subcore_barrier()

        def body(i, carry):
            off = sid * per_t + i * chunk
            pltpu.sync_copy(idx_hbm.at[pl.ds(off, chunk)], idx_v)
            pltpu.sync_copy(vals_hbm.at[pl.ds(off, chunk)], val_v)
            pltpu.sync_copy(val_v, accum.at[idx_v], add=True)
            return carry

        lax.fori_loop(0, nch, body, 0)
        plsc.subcore_barrier()
        o0 = cid * half + sid * rows_out
        pltpu.sync_copy(accum.at[pl.ds(o0, rows_out)],
                        out_hbm.at[pl.ds(o0, rows_out)])

    return k(values, idx, prev)


def _wspec(shp):
    return pl.BlockSpec(shp, lambda i: (0, 0))


def _tc_round1(cfct, evl, B, b1t, W2, b2, W3, b3, W4, b4, T):
    E = evl.shape[0]
    nblk = E // T

    def body(cf_ref, ct_ref, ev_ref, bmat, bb1, w2, bb2, w3, bb3, w4, bb4,
             msg_ref, ef_ref):
        cfv, ctv, evv = cf_ref[...], ct_ref[...], ev_ref[...]
        d = cfv - ctv
        lane = lax.broadcasted_iota(jnp.int32, (1, EFD), 1)
        p1 = jnp.where(lane == 3, evv, jnp.abs(d))
        ef = jnp.concatenate([p1, cfv * ctv, d * evv], axis=1)
        ef_ref[...] = ef
        h = jnp.tanh(ef @ bmat[...] + bb1[...])
        h = jnp.tanh(h @ w2[...] + bb2[...])
        h = jnp.tanh(h @ w3[...] + bb3[...])
        m = jnp.tanh(h @ w4[...] + bb4[...])
        msg_ref[...] = jnp.concatenate(
            [m, jnp.zeros((m.shape[0], SD - STATE), jnp.float32)], axis=1)

    return pl.pallas_call(
        body,
        grid=(nblk,),
        in_specs=[
            pl.BlockSpec((T, EFD), lambda i: (i, 0)),
            pl.BlockSpec((T, EFD), lambda i: (i + nblk, 0)),
            pl.BlockSpec((T, EFD), lambda i: (i, 0)),
            _wspec((SD, 40)), _wspec((1, 40)),
            _wspec((40, 150)), _wspec((1, 150)),
            _wspec((150, 40)), _wspec((1, 40)),
            _wspec((40, STATE)), _wspec((1, STATE)),
        ],
        out_specs=[
            pl.BlockSpec((T, SD), lambda i: (i, 0)),
            pl.BlockSpec((T, SD), lambda i: (i, 0)),
        ],
        out_shape=[
            jax.ShapeDtypeStruct((E, SD), jnp.float32),
            jax.ShapeDtypeStruct((E, SD), jnp.float32),
        ],
        compiler_params=pltpu.CompilerParams(
            dimension_semantics=("parallel",)),
    )(cfct, cfct, evl, B, b1t, W2, b2, W3, b3, W4, b4)


def _tc_mlp(gathered, ef, W1a, B, b1t, W2, b2, W3, b3, W4, b4, T):
    E = gathered.shape[0]

    def body(g_ref, ef_ref, w1a, bmat, bb1, w2, bb2, w3, bb3, w4, bb4,
             out_ref):
        z = g_ref[...] @ w1a[...] + ef_ref[...] @ bmat[...] + bb1[...]
        h = jnp.tanh(z)
        h = jnp.tanh(h @ w2[...] + bb2[...])
        h = jnp.tanh(h @ w3[...] + bb3[...])
        m = jnp.tanh(h @ w4[...] + bb4[...])
        out_ref[...] = jnp.concatenate(
            [m, jnp.zeros((m.shape[0], SD - STATE), jnp.float32)], axis=1)

    return pl.pallas_call(
        body,
        grid=(E // T,),
        in_specs=[
            pl.BlockSpec((T, SD), lambda i: (i, 0)),
            pl.BlockSpec((T, SD), lambda i: (i, 0)),
            _wspec((SD, 40)), _wspec((SD, 40)), _wspec((1, 40)),
            _wspec((40, 150)), _wspec((1, 150)),
            _wspec((150, 40)), _wspec((1, 40)),
            _wspec((40, STATE)), _wspec((1, STATE)),
        ],
        out_specs=pl.BlockSpec((T, SD), lambda i: (i, 0)),
        out_shape=jax.ShapeDtypeStruct((E, SD), jnp.float32),
        compiler_params=pltpu.CompilerParams(
            dimension_semantics=("parallel",)),
    )(gathered, ef, W1a, B, b1t, W2, b2, W3, b3, W4, b4)


def _tc_outnet(gstate, Wo, bo):
    G = gstate.shape[0]

    def body(g_ref, wo, bo_, out_ref):
        ev = g_ref[...] @ wo[...] + bo_[...]
        mu = ev[:, 0:1]
        sg = ev[:, 1:2]
        sp = jnp.maximum(sg, 0.0) + jnp.log1p(jnp.exp(-jnp.abs(sg)))
        out_ref[...] = jnp.concatenate([mu, sp], axis=1)

    return pl.pallas_call(
        body,
        grid=(1,),
        in_specs=[
            pl.BlockSpec((G, SD), lambda i: (0, 0)),
            _wspec((SD, 2)), _wspec((1, 2)),
        ],
        out_specs=pl.BlockSpec((G, 2), lambda i: (0, 0)),
        out_shape=jax.ShapeDtypeStruct((G, 2), jnp.float32),
    )(gstate, Wo, bo)


def kernel(node_coordinates, edge_lengths, edge_vectors, node_from, node_to,
           node_graph_index, num_nodes, num_graphs,
           bn_gamma, bn_beta, bn_mean, bn_var,
           W1, b1, W2, b2, W3, b3, W4, b4, Wo, bo):
    E = node_from.shape[0]
    N = node_coordinates.shape[0]
    NP = _round_up(N, 1024)
    G = _round_up(NG, 64)
    T = _pick_chunk(E, 4096)

    nf = node_from.astype(jnp.int32)
    nt = node_to.astype(jnp.int32)
    gidx = node_graph_index.astype(jnp.int32)

    s = bn_gamma * lax.rsqrt(bn_var + 1e-5)
    t = bn_beta - bn_mean * s
    W1s = W1 * s[:, None]
    b1t = (b1 + t @ W1).reshape(1, -1)
    W1a = jnp.pad(W1s[:STATE], ((0, SD - STATE), (0, 0)))
    B = jnp.zeros((SD, 40), jnp.float32)
    B = B.at[0:3].set(jnp.broadcast_to(W1s[STATE + 1], (3, 40)))
    B = B.at[3].set(W1s[STATE + 0])
    B = B.at[8:11].set(jnp.broadcast_to(W1s[STATE + 2], (3, 40)))
    B = B.at[16:19].set(jnp.broadcast_to(W1s[STATE + 3], (3, 40)))
    b2r, b3r, b4r = b2.reshape(1, -1), b3.reshape(1, -1), b4.reshape(1, -1)
    Wo24 = jnp.pad(Wo, ((0, SD - STATE), (0, 0)))
    bor = bo.reshape(1, -1)

    coords8 = jnp.pad(node_coordinates, ((0, 0), (0, EFD - 3)))
    evl = jnp.pad(
        jnp.concatenate([edge_vectors, edge_lengths], axis=1),
        ((0, 0), (0, EFD - 4)))
    cfct = _gather_rows(coords8, jnp.concatenate([nf, nt]))

    msg, ef = _tc_round1(cfct, evl, B, b1t, W2, b2r, W3, b3r, W4, b4r, T)

    state = jnp.zeros((NP, SD), jnp.float32)
    for r in range(3):
        state = _scatter_add_rows(msg, nt, state)
        if r < 2:
            gathered = _gather_rows(state, nf)
            msg = _tc_mlp(gathered, ef, W1a, B, b1t,
                          W2, b2r, W3, b3r, W4, b4r, T)

    gidx_pad = jnp.concatenate(
        [gidx, jnp.full((NP - N,), G - 1, jnp.int32)])
    gstate = _scatter_add_rows(state, gidx_pad,
                               jnp.zeros((G, SD), jnp.float32))

    out = _tc_outnet(gstate, Wo24, bor)
    return out[:NG]


# speedup vs baseline: 3.5963x; 1.2121x over previous
import functools

import jax
import jax.numpy as jnp
from jax import lax
from jax.experimental import pallas as pl
from jax.experimental.pallas import tpu as pltpu
from jax.experimental.pallas import tpu_sc as plsc

STATE = 20
SD = 24
EFD = 8
NG = 500
NCORES = 2
NSUB = 16
NW = NCORES * NSUB


def _sc_mesh():
    return plsc.VectorSubcoreMesh(core_axis_name="c", subcore_axis_name="s")


def _round_up(x, m):
    return (x + m - 1) // m * m


def _pick_chunk(total, cap):
    c = min(total, max(cap, 8)) // 8 * 8
    while c > 8 and total % c != 0:
        c -= 8
    return c


def _gather_rows(table, idx):
    M = idx.shape[0]
    _, D = table.shape
    per_w = M // NW
    chunk = _pick_chunk(per_w, 440_000 // (8 * (D + 1)))
    nch = per_w // chunk

    @functools.partial(
        pl.kernel,
        out_type=jax.ShapeDtypeStruct((M, D), jnp.float32),
        mesh=_sc_mesh(),
        scratch_types=[
            pltpu.VMEM((chunk,), jnp.int32),
            pltpu.VMEM((chunk,), jnp.int32),
            pltpu.VMEM((chunk, D), jnp.float32),
            pltpu.VMEM((chunk, D), jnp.float32),
            pltpu.SemaphoreType.DMA,
            pltpu.SemaphoreType.DMA,
            pltpu.SemaphoreType.DMA,
        ],
        compiler_params=pltpu.CompilerParams(use_tc_tiling_on_sc=False),
    )
    def k(table_hbm, idx_hbm, out_hbm, idx_a, idx_b, rows_a, rows_b,
          isem, gsem, osem):
        wid = lax.axis_index("s") * NCORES + lax.axis_index("c")
        base = wid * per_w
        idx_bufs = [idx_a, idx_b]
        row_bufs = [rows_a, rows_b]
        idx_d = [None] * nch
        out_d = [None] * nch
        idx_d[0] = pltpu.async_copy(
            idx_hbm.at[pl.ds(base, chunk)], idx_bufs[0], isem)
        for i in range(nch):
            p = i % 2
            idx_d[i].wait()
            if i + 1 < nch:
                idx_d[i + 1] = pltpu.async_copy(
                    idx_hbm.at[pl.ds(base + (i + 1) * chunk, chunk)],
                    idx_bufs[(i + 1) % 2], isem)
            if i >= 2:
                out_d[i - 2].wait()
            pltpu.async_copy(
                table_hbm.at[idx_bufs[p]], row_bufs[p], gsem).wait()
            out_d[i] = pltpu.async_copy(
                row_bufs[p], out_hbm.at[pl.ds(base + i * chunk, chunk)],
                osem)
        if nch >= 2:
            out_d[nch - 2].wait()
        out_d[nch - 1].wait()

    return k(table, idx)


def _scatter_add_rows(values, idx, prev):
    M, D = values.shape
    P = prev.shape[0]
    per_t = M // NSUB
    chunk = _pick_chunk(per_t, 440_000 // (8 * (D + 1)))
    nch = per_t // chunk
    rows_init = P // NSUB
    half = P // NCORES
    rows_out = half // NSUB

    @functools.partial(
        pl.kernel,
        out_type=jax.ShapeDtypeStruct((P, D), jnp.float32),
        mesh=_sc_mesh(),
        scratch_types=[
            pltpu.VMEM((chunk,), jnp.int32),
            pltpu.VMEM((chunk, D), jnp.float32),
            pltpu.VMEM_SHARED((P, D), jnp.float32),
            pltpu.SemaphoreType.DMA,
        ],
        compiler_params=pltpu.CompilerParams(use_tc_tiling_on_sc=False),
    )
    def k(vals_hbm, idx_hbm, prev_hbm, out_hbm, idx_v, val_v, accum, sem):
        cid = lax.axis_index("c")
        sid = lax.axis_index("s")
        r0 = sid * rows_init
        pltpu.sync_copy(prev_hbm.at[pl.ds(r0, rows_init)],
                        accum.at[pl.ds(r0, rows_init)])
        plsc.subcore_barrier()

        def body(i, carry):
            off = sid * per_t + i * chunk
            pltpu.sync_copy(idx_hbm.at[pl.ds(off, chunk)], idx_v)
            pltpu.sync_copy(vals_hbm.at[pl.ds(off, chunk)], val_v)
            pltpu.sync_copy(val_v, accum.at[idx_v], add=True)
            return carry

        lax.fori_loop(0, nch, body, 0)
        plsc.subcore_barrier()
        o0 = cid * half + sid * rows_out
        pltpu.sync_copy(accum.at[pl.ds(o0, rows_out)],
                        out_hbm.at[pl.ds(o0, rows_out)])

    return k(values, idx, prev)


def _wspec(shp):
    return pl.BlockSpec(shp, lambda i: (0, 0))


def _tc_round1(cfct, evl, B, b1t, W2, b2, W3, b3, W4, b4, T):
    E = evl.shape[0]
    nblk = E // T

    def body(cf_ref, ct_ref, ev_ref, bmat, bb1, w2, bb2, w3, bb3, w4, bb4,
             msg_ref, ef_ref):
        cfv, ctv, evv = cf_ref[...], ct_ref[...], ev_ref[...]
        ev8 = jnp.concatenate(
            [evv, jnp.zeros((evv.shape[0], EFD - 4), jnp.float32)], axis=1)
        d = cfv - ctv
        lane = lax.broadcasted_iota(jnp.int32, (1, EFD), 1)
        p1 = jnp.where(lane == 3, ev8, jnp.abs(d))
        ef = jnp.concatenate([p1, cfv * ctv, d * ev8], axis=1)
        ef_ref[...] = ef
        h = jnp.tanh(ef @ bmat[...] + bb1[...])
        h = jnp.tanh(h @ w2[...] + bb2[...])
        h = jnp.tanh(h @ w3[...] + bb3[...])
        m = jnp.tanh(h @ w4[...] + bb4[...])
        msg_ref[...] = jnp.concatenate(
            [m, jnp.zeros((m.shape[0], SD - STATE), jnp.float32)], axis=1)

    return pl.pallas_call(
        body,
        grid=(nblk,),
        in_specs=[
            pl.BlockSpec((T, EFD), lambda i: (i, 0)),
            pl.BlockSpec((T, EFD), lambda i: (i + nblk, 0)),
            pl.BlockSpec((T, 4), lambda i: (i, 0)),
            _wspec((SD, 40)), _wspec((1, 40)),
            _wspec((40, 150)), _wspec((1, 150)),
            _wspec((150, 40)), _wspec((1, 40)),
            _wspec((40, STATE)), _wspec((1, STATE)),
        ],
        out_specs=[
            pl.BlockSpec((T, SD), lambda i: (i, 0)),
            pl.BlockSpec((T, SD), lambda i: (i, 0)),
        ],
        out_shape=[
            jax.ShapeDtypeStruct((E, SD), jnp.float32),
            jax.ShapeDtypeStruct((E, SD), jnp.float32),
        ],
        compiler_params=pltpu.CompilerParams(
            dimension_semantics=("parallel",)),
    )(cfct, cfct, evl, B, b1t, W2, b2, W3, b3, W4, b4)


def _tc_mlp(gathered, ef, W1a, B, b1t, W2, b2, W3, b3, W4, b4, T):
    E = gathered.shape[0]

    def body(g_ref, ef_ref, w1a, bmat, bb1, w2, bb2, w3, bb3, w4, bb4,
             out_ref):
        z = g_ref[...] @ w1a[...] + ef_ref[...] @ bmat[...] + bb1[...]
        h = jnp.tanh(z)
        h = jnp.tanh(h @ w2[...] + bb2[...])
        h = jnp.tanh(h @ w3[...] + bb3[...])
        m = jnp.tanh(h @ w4[...] + bb4[...])
        out_ref[...] = jnp.concatenate(
            [m, jnp.zeros((m.shape[0], SD - STATE), jnp.float32)], axis=1)

    return pl.pallas_call(
        body,
        grid=(E // T,),
        in_specs=[
            pl.BlockSpec((T, SD), lambda i: (i, 0)),
            pl.BlockSpec((T, SD), lambda i: (i, 0)),
            _wspec((SD, 40)), _wspec((SD, 40)), _wspec((1, 40)),
            _wspec((40, 150)), _wspec((1, 150)),
            _wspec((150, 40)), _wspec((1, 40)),
            _wspec((40, STATE)), _wspec((1, STATE)),
        ],
        out_specs=pl.BlockSpec((T, SD), lambda i: (i, 0)),
        out_shape=jax.ShapeDtypeStruct((E, SD), jnp.float32),
        compiler_params=pltpu.CompilerParams(
            dimension_semantics=("parallel",)),
    )(gathered, ef, W1a, B, b1t, W2, b2, W3, b3, W4, b4)


def _tc_outnet(gstate, Wo, bo):
    G = gstate.shape[0]

    def body(g_ref, wo, bo_, out_ref):
        ev = g_ref[...] @ wo[...] + bo_[...]
        mu = ev[:, 0:1]
        sg = ev[:, 1:2]
        sp = jnp.maximum(sg, 0.0) + jnp.log1p(jnp.exp(-jnp.abs(sg)))
        out_ref[...] = jnp.concatenate([mu, sp], axis=1)

    return pl.pallas_call(
        body,
        grid=(1,),
        in_specs=[
            pl.BlockSpec((G, SD), lambda i: (0, 0)),
            _wspec((SD, 2)), _wspec((1, 2)),
        ],
        out_specs=pl.BlockSpec((G, 2), lambda i: (0, 0)),
        out_shape=jax.ShapeDtypeStruct((G, 2), jnp.float32),
    )(gstate, Wo, bo)


def kernel(node_coordinates, edge_lengths, edge_vectors, node_from, node_to,
           node_graph_index, num_nodes, num_graphs,
           bn_gamma, bn_beta, bn_mean, bn_var,
           W1, b1, W2, b2, W3, b3, W4, b4, Wo, bo):
    E = node_from.shape[0]
    N = node_coordinates.shape[0]
    NP = _round_up(N, 1024)
    G = _round_up(NG, 64)
    T = _pick_chunk(E, 8000)

    nf = node_from.astype(jnp.int32)
    nt = node_to.astype(jnp.int32)
    gidx = node_graph_index.astype(jnp.int32)

    s = bn_gamma * lax.rsqrt(bn_var + 1e-5)
    t = bn_beta - bn_mean * s
    W1s = W1 * s[:, None]
    b1t = (b1 + t @ W1).reshape(1, -1)
    W1a = jnp.pad(W1s[:STATE], ((0, SD - STATE), (0, 0)))
    B = jnp.zeros((SD, 40), jnp.float32)
    B = B.at[0:3].set(jnp.broadcast_to(W1s[STATE + 1], (3, 40)))
    B = B.at[3].set(W1s[STATE + 0])
    B = B.at[8:11].set(jnp.broadcast_to(W1s[STATE + 2], (3, 40)))
    B = B.at[16:19].set(jnp.broadcast_to(W1s[STATE + 3], (3, 40)))
    b2r, b3r, b4r = b2.reshape(1, -1), b3.reshape(1, -1), b4.reshape(1, -1)
    Wo24 = jnp.pad(Wo, ((0, SD - STATE), (0, 0)))
    bor = bo.reshape(1, -1)

    coords8 = jnp.pad(node_coordinates, ((0, 0), (0, EFD - 3)))
    evl = jnp.concatenate([edge_vectors, edge_lengths], axis=1)
    cfct = _gather_rows(coords8, jnp.concatenate([nf, nt]))

    msg, ef = _tc_round1(cfct, evl, B, b1t, W2, b2r, W3, b3r, W4, b4r, T)

    state = jnp.zeros((NP, SD), jnp.float32)
    for r in range(3):
        state = _scatter_add_rows(msg, nt, state)
        if r < 2:
            gathered = _gather_rows(state, nf)
            msg = _tc_mlp(gathered, ef, W1a, B, b1t,
                          W2, b2r, W3, b3r, W4, b4r, T)

    gidx_pad = jnp.concatenate(
        [gidx, jnp.full((NP - N,), G - 1, jnp.int32)])
    gstate = _scatter_add_rows(state, gidx_pad,
                               jnp.zeros((G, SD), jnp.float32))

    out = _tc_outnet(gstate, Wo24, bor)
    return out[:NG]


# speedup vs baseline: 3.6934x; 1.0270x over previous
import functools

import jax
import jax.numpy as jnp
from jax import lax
from jax.experimental import pallas as pl
from jax.experimental.pallas import tpu as pltpu
from jax.experimental.pallas import tpu_sc as plsc

STATE = 20
SD = 24
EFD = 8
NG = 500
NCORES = 2
NSUB = 16
NW = NCORES * NSUB


def _sc_mesh():
    return plsc.VectorSubcoreMesh(core_axis_name="c", subcore_axis_name="s")


def _round_up(x, m):
    return (x + m - 1) // m * m


def _pick_chunk(total, cap):
    c = min(total, max(cap, 8)) // 8 * 8
    while c > 8 and total % c != 0:
        c -= 8
    return c


def _gather_rows(table, idx):
    M = idx.shape[0]
    _, D = table.shape
    per_w = M // NW
    chunk = _pick_chunk(per_w, 440_000 // (8 * (D + 1)))
    nch = per_w // chunk

    @functools.partial(
        pl.kernel,
        out_type=jax.ShapeDtypeStruct((M, D), jnp.float32),
        mesh=_sc_mesh(),
        scratch_types=[
            pltpu.VMEM((chunk,), jnp.int32),
            pltpu.VMEM((chunk,), jnp.int32),
            pltpu.VMEM((chunk, D), jnp.float32),
            pltpu.VMEM((chunk, D), jnp.float32),
            pltpu.SemaphoreType.DMA,
            pltpu.SemaphoreType.DMA,
            pltpu.SemaphoreType.DMA,
        ],
        compiler_params=pltpu.CompilerParams(use_tc_tiling_on_sc=False),
    )
    def k(table_hbm, idx_hbm, out_hbm, idx_a, idx_b, rows_a, rows_b,
          isem, gsem, osem):
        wid = lax.axis_index("s") * NCORES + lax.axis_index("c")
        base = wid * per_w
        idx_bufs = [idx_a, idx_b]
        row_bufs = [rows_a, rows_b]
        idx_d = [None] * nch
        out_d = [None] * nch
        idx_d[0] = pltpu.async_copy(
            idx_hbm.at[pl.ds(base, chunk)], idx_bufs[0], isem)
        for i in range(nch):
            p = i % 2
            idx_d[i].wait()
            if i + 1 < nch:
                idx_d[i + 1] = pltpu.async_copy(
                    idx_hbm.at[pl.ds(base + (i + 1) * chunk, chunk)],
                    idx_bufs[(i + 1) % 2], isem)
            if i >= 2:
                out_d[i - 2].wait()
            pltpu.async_copy(
                table_hbm.at[idx_bufs[p]], row_bufs[p], gsem).wait()
            out_d[i] = pltpu.async_copy(
                row_bufs[p], out_hbm.at[pl.ds(base + i * chunk, chunk)],
                osem)
        if nch >= 2:
            out_d[nch - 2].wait()
        out_d[nch - 1].wait()

    return k(table, idx)


def _scatter_add_rows(values, idx, prev):
    M, D = values.shape
    P = prev.shape[0]
    per_t = M // NSUB
    chunk = _pick_chunk(per_t, 220_000 // (8 * (D + 1)))
    nch = per_t // chunk
    rows_init = P // NSUB
    half = P // NCORES
    rows_out = half // NSUB

    @functools.partial(
        pl.kernel,
        out_type=jax.ShapeDtypeStruct((P, D), jnp.float32),
        mesh=_sc_mesh(),
        scratch_types=[
            pltpu.VMEM((chunk,), jnp.int32),
            pltpu.VMEM((chunk,), jnp.int32),
            pltpu.VMEM((chunk, D), jnp.float32),
            pltpu.VMEM((chunk, D), jnp.float32),
            pltpu.VMEM_SHARED((P, D), jnp.float32),
            pltpu.SemaphoreType.DMA,
            pltpu.SemaphoreType.DMA,
            pltpu.SemaphoreType.DMA,
        ],
        compiler_params=pltpu.CompilerParams(use_tc_tiling_on_sc=False),
    )
    def k(vals_hbm, idx_hbm, prev_hbm, out_hbm, idx_a, idx_b, val_a, val_b,
          accum, isem, vsem, ssem):
        cid = lax.axis_index("c")
        sid = lax.axis_index("s")
        idx_bufs = [idx_a, idx_b]
        val_bufs = [val_a, val_b]
        base = sid * per_t
        idx_d = [None] * nch
        val_d = [None] * nch
        sc_d = [None] * nch
        idx_d[0] = pltpu.async_copy(
            idx_hbm.at[pl.ds(base, chunk)], idx_bufs[0], isem)
        val_d[0] = pltpu.async_copy(
            vals_hbm.at[pl.ds(base, chunk)], val_bufs[0], vsem)
        r0 = sid * rows_init
        pltpu.sync_copy(prev_hbm.at[pl.ds(r0, rows_init)],
                        accum.at[pl.ds(r0, rows_init)])
        plsc.subcore_barrier()

        for i in range(nch):
            p = i % 2
            idx_d[i].wait()
            val_d[i].wait()
            if i >= 1:
                sc_d[i - 1].wait()
            if i + 1 < nch:
                off = base + (i + 1) * chunk
                idx_d[i + 1] = pltpu.async_copy(
                    idx_hbm.at[pl.ds(off, chunk)], idx_bufs[(i + 1) % 2],
                    isem)
                val_d[i + 1] = pltpu.async_copy(
                    vals_hbm.at[pl.ds(off, chunk)], val_bufs[(i + 1) % 2],
                    vsem)
            sc_d[i] = pltpu.async_copy(
                val_bufs[p], accum.at[idx_bufs[p]], ssem, add=True)
        sc_d[nch - 1].wait()
        plsc.subcore_barrier()
        o0 = cid * half + sid * rows_out
        pltpu.sync_copy(accum.at[pl.ds(o0, rows_out)],
                        out_hbm.at[pl.ds(o0, rows_out)])

    return k(values, idx, prev)


def _wspec(shp):
    return pl.BlockSpec(shp, lambda i: (0, 0))


def _tc_round1(cfct, evl, B, b1t, W2, b2, W3, b3, W4, b4, T):
    E = evl.shape[0]
    nblk = E // T

    def body(cf_ref, ct_ref, ev_ref, bmat, bb1, w2, bb2, w3, bb3, w4, bb4,
             msg_ref, ef_ref):
        cfv, ctv, evv = cf_ref[...], ct_ref[...], ev_ref[...]
        ev8 = jnp.concatenate(
            [evv, jnp.zeros((evv.shape[0], EFD - 4), jnp.float32)], axis=1)
        d = cfv - ctv
        lane = lax.broadcasted_iota(jnp.int32, (1, EFD), 1)
        p1 = jnp.where(lane == 3, ev8, jnp.abs(d))
        ef = jnp.concatenate([p1, cfv * ctv, d * ev8], axis=1)
        ef_ref[...] = ef
        h = jnp.tanh(ef @ bmat[...] + bb1[...])
        h = jnp.tanh(h @ w2[...] + bb2[...])
        h = jnp.tanh(h @ w3[...] + bb3[...])
        m = jnp.tanh(h @ w4[...] + bb4[...])
        msg_ref[...] = jnp.concatenate(
            [m, jnp.zeros((m.shape[0], SD - STATE), jnp.float32)], axis=1)

    return pl.pallas_call(
        body,
        grid=(nblk,),
        in_specs=[
            pl.BlockSpec((T, EFD), lambda i: (i, 0)),
            pl.BlockSpec((T, EFD), lambda i: (i + nblk, 0)),
            pl.BlockSpec((T, 4), lambda i: (i, 0)),
            _wspec((SD, 40)), _wspec((1, 40)),
            _wspec((40, 150)), _wspec((1, 150)),
            _wspec((150, 40)), _wspec((1, 40)),
            _wspec((40, STATE)), _wspec((1, STATE)),
        ],
        out_specs=[
            pl.BlockSpec((T, SD), lambda i: (i, 0)),
            pl.BlockSpec((T, SD), lambda i: (i, 0)),
        ],
        out_shape=[
            jax.ShapeDtypeStruct((E, SD), jnp.float32),
            jax.ShapeDtypeStruct((E, SD), jnp.float32),
        ],
        compiler_params=pltpu.CompilerParams(
            dimension_semantics=("parallel",)),
    )(cfct, cfct, evl, B, b1t, W2, b2, W3, b3, W4, b4)


def _tc_mlp(gathered, ef, W1a, B, b1t, W2, b2, W3, b3, W4, b4, T):
    E = gathered.shape[0]

    def body(g_ref, ef_ref, w1a, bmat, bb1, w2, bb2, w3, bb3, w4, bb4,
             out_ref):
        z = g_ref[...] @ w1a[...] + ef_ref[...] @ bmat[...] + bb1[...]
        h = jnp.tanh(z)
        h = jnp.tanh(h @ w2[...] + bb2[...])
        h = jnp.tanh(h @ w3[...] + bb3[...])
        m = jnp.tanh(h @ w4[...] + bb4[...])
        out_ref[...] = jnp.concatenate(
            [m, jnp.zeros((m.shape[0], SD - STATE), jnp.float32)], axis=1)

    return pl.pallas_call(
        body,
        grid=(E // T,),
        in_specs=[
            pl.BlockSpec((T, SD), lambda i: (i, 0)),
            pl.BlockSpec((T, SD), lambda i: (i, 0)),
            _wspec((SD, 40)), _wspec((SD, 40)), _wspec((1, 40)),
            _wspec((40, 150)), _wspec((1, 150)),
            _wspec((150, 40)), _wspec((1, 40)),
            _wspec((40, STATE)), _wspec((1, STATE)),
        ],
        out_specs=pl.BlockSpec((T, SD), lambda i: (i, 0)),
        out_shape=jax.ShapeDtypeStruct((E, SD), jnp.float32),
        compiler_params=pltpu.CompilerParams(
            dimension_semantics=("parallel",)),
    )(gathered, ef, W1a, B, b1t, W2, b2, W3, b3, W4, b4)


def _tc_outnet(gstate, Wo, bo):
    G = gstate.shape[0]

    def body(g_ref, wo, bo_, out_ref):
        ev = g_ref[...] @ wo[...] + bo_[...]
        mu = ev[:, 0:1]
        sg = ev[:, 1:2]
        sp = jnp.maximum(sg, 0.0) + jnp.log1p(jnp.exp(-jnp.abs(sg)))
        out_ref[...] = jnp.concatenate([mu, sp], axis=1)

    return pl.pallas_call(
        body,
        grid=(1,),
        in_specs=[
            pl.BlockSpec((G, SD), lambda i: (0, 0)),
            _wspec((SD, 2)), _wspec((1, 2)),
        ],
        out_specs=pl.BlockSpec((G, 2), lambda i: (0, 0)),
        out_shape=jax.ShapeDtypeStruct((G, 2), jnp.float32),
    )(gstate, Wo, bo)


def kernel(node_coordinates, edge_lengths, edge_vectors, node_from, node_to,
           node_graph_index, num_nodes, num_graphs,
           bn_gamma, bn_beta, bn_mean, bn_var,
           W1, b1, W2, b2, W3, b3, W4, b4, Wo, bo):
    E = node_from.shape[0]
    N = node_coordinates.shape[0]
    NP = _round_up(N, 1024)
    G = _round_up(NG, 64)
    T = _pick_chunk(E, 8000)

    nf = node_from.astype(jnp.int32)
    nt = node_to.astype(jnp.int32)
    gidx = node_graph_index.astype(jnp.int32)

    s = bn_gamma * lax.rsqrt(bn_var + 1e-5)
    t = bn_beta - bn_mean * s
    W1s = W1 * s[:, None]
    b1t = (b1 + t @ W1).reshape(1, -1)
    W1a = jnp.pad(W1s[:STATE], ((0, SD - STATE), (0, 0)))
    B = jnp.zeros((SD, 40), jnp.float32)
    B = B.at[0:3].set(jnp.broadcast_to(W1s[STATE + 1], (3, 40)))
    B = B.at[3].set(W1s[STATE + 0])
    B = B.at[8:11].set(jnp.broadcast_to(W1s[STATE + 2], (3, 40)))
    B = B.at[16:19].set(jnp.broadcast_to(W1s[STATE + 3], (3, 40)))
    b2r, b3r, b4r = b2.reshape(1, -1), b3.reshape(1, -1), b4.reshape(1, -1)
    Wo24 = jnp.pad(Wo, ((0, SD - STATE), (0, 0)))
    bor = bo.reshape(1, -1)

    coords8 = jnp.pad(node_coordinates, ((0, 0), (0, EFD - 3)))
    evl = jnp.concatenate([edge_vectors, edge_lengths], axis=1)
    cfct = _gather_rows(coords8, jnp.concatenate([nf, nt]))

    msg, ef = _tc_round1(cfct, evl, B, b1t, W2, b2r, W3, b3r, W4, b4r, T)

    state = jnp.zeros((NP, SD), jnp.float32)
    for r in range(3):
        state = _scatter_add_rows(msg, nt, state)
        if r < 2:
            gathered = _gather_rows(state, nf)
            msg = _tc_mlp(gathered, ef, W1a, B, b1t,
                          W2, b2r, W3, b3r, W4, b4r, T)

    gidx_pad = jnp.concatenate(
        [gidx, jnp.full((NP - N,), G - 1, jnp.int32)])
    gstate = _scatter_add_rows(state, gidx_pad,
                               jnp.zeros((G, SD), jnp.float32))

    out = _tc_outnet(gstate, Wo24, bor)
    return out[:NG]
